# baseline (speedup 1.0000x reference)
import functools

import jax
import jax.numpy as jnp
from jax import lax
from jax.experimental import pallas as pl
from jax.experimental.pallas import tpu as pltpu
from jax.experimental.pallas import tpu_sc as plsc

_NC = 2
_NS = 16
_NW = _NC * _NS

_TOPK = 2


def _routing_body(x_ref, wg_ref, bg_ref, w_ref, i_ref):
    logits = jnp.dot(x_ref[...], wg_ref[...],
                     preferred_element_type=jnp.float32) + bg_ref[0][None, :]
    bn, e = logits.shape
    iota = lax.broadcasted_iota(jnp.int32, (bn, e), 1)
    m1 = jnp.max(logits, axis=1, keepdims=True)
    i1 = jnp.min(jnp.where(logits == m1, iota, e), axis=1, keepdims=True)
    masked = jnp.where(iota == i1, -jnp.inf, logits)
    m2 = jnp.max(masked, axis=1, keepdims=True)
    i2 = jnp.min(jnp.where(masked == m2, iota, e), axis=1, keepdims=True)
    z = jnp.exp(m2 - m1)
    w2 = z / (1.0 + z)
    w_ref[...] = jnp.concatenate([1.0 - w2, w2], axis=1)
    i_ref[...] = jnp.concatenate([i1, i2], axis=1).astype(jnp.int32)


def _route(x, Wg, bg):
    n, d = x.shape
    e = Wg.shape[1]
    bn = 256
    return pl.pallas_call(
        _routing_body,
        grid=(n // bn,),
        in_specs=[
            pl.BlockSpec((bn, d), lambda i: (i, 0)),
            pl.BlockSpec((d, e), lambda i: (0, 0)),
            pl.BlockSpec((1, e), lambda i: (0, 0)),
        ],
        out_specs=[
            pl.BlockSpec((bn, _TOPK), lambda i: (i, 0)),
            pl.BlockSpec((bn, _TOPK), lambda i: (i, 0)),
        ],
        out_shape=[
            jax.ShapeDtypeStruct((n, _TOPK), jnp.float32),
            jax.ShapeDtypeStruct((n, _TOPK), jnp.int32),
        ],
    )(x, Wg, bg.reshape(1, e))


def _sc_gather(x, gidx, p_rows):
    n, d = x.shape
    per_w = p_rows // _NW
    chunks = []
    off = 0
    while off < per_w:
        sz = min(64, per_w - off)
        chunks.append((off, sz))
        off += sz
    mesh = plsc.VectorSubcoreMesh(core_axis_name="c", subcore_axis_name="s")

    @functools.partial(
        pl.kernel,
        mesh=mesh,
        out_type=jax.ShapeDtypeStruct((p_rows, d), jnp.float32),
        scratch_types=(
            [pltpu.VMEM((sz,), jnp.int32) for _, sz in chunks]
            + [pltpu.VMEM((64, d), jnp.float32),
               pltpu.VMEM((64, d), jnp.float32),
               pltpu.SemaphoreType.DMA,
               pltpu.SemaphoreType.DMA,
               pltpu.SemaphoreType.DMA,
               pltpu.SemaphoreType.DMA]
        ),
    )
    def k(x_hbm, gidx_hbm, out_hbm, *scratch):
        idx_refs = scratch[:len(chunks)]
        rows0, rows1, gs0, gs1, ws0, ws1 = scratch[len(chunks):]
        wid = lax.axis_index("s") * _NC + lax.axis_index("c")
        base = wid * per_w
        for (off, sz), iv in zip(chunks, idx_refs):
            pltpu.sync_copy(gidx_hbm.at[pl.ds(base + off, sz)], iv)
        bufs = (rows0, rows1)
        gsems = (gs0, gs1)
        wsems = (ws0, ws1)
        gathers = []
        writes = []
        for i, (off, sz) in enumerate(chunks):
            b = i % 2
            if i >= 2:
                writes[i - 2].wait()
            gathers.append(pltpu.async_copy(
                x_hbm.at[idx_refs[i]],
                bufs[b].at[pl.ds(0, sz)], gsems[b]))
            gathers[i].wait()
            writes.append(pltpu.async_copy(
                bufs[b].at[pl.ds(0, sz)],
                out_hbm.at[pl.ds(base + off, sz)], wsems[b]))
        for w in writes[max(0, len(chunks) - 2):]:
            w.wait()

    return k(x, gidx)


def _ffn_body(te_ref, valid_ref, xg_ref, w1_ref, b1_ref, w2_ref, b2_ref,
              y_ref):
    i = pl.program_id(0)

    @pl.when(valid_ref[i] != 0)
    def _():
        xb = xg_ref[...].astype(jnp.bfloat16)
        h = jnp.dot(xb, w1_ref[0],
                    preferred_element_type=jnp.float32) + b1_ref[0]
        h = jnp.maximum(h, 0.0).astype(jnp.bfloat16)
        y_ref[...] = jnp.dot(h, w2_ref[0],
                             preferred_element_type=jnp.float32) + b2_ref[0]


def _ffn(xg, te, valid, W1, b1, W2, b2, tile, nt):
    p_rows, d = xg.shape
    e, _, h = W1.shape
    grid_spec = pltpu.PrefetchScalarGridSpec(
        num_scalar_prefetch=2,
        grid=(nt,),
        in_specs=[
            pl.BlockSpec((tile, d), lambda i, te, v: (i, 0)),
            pl.BlockSpec((1, d, h), lambda i, te, v: (te[i], 0, 0)),
            pl.BlockSpec((1, 1, h), lambda i, te, v: (te[i], 0, 0)),
            pl.BlockSpec((1, h, d), lambda i, te, v: (te[i], 0, 0)),
            pl.BlockSpec((1, 1, d), lambda i, te, v: (te[i], 0, 0)),
        ],
        out_specs=pl.BlockSpec((tile, d), lambda i, te, v: (i, 0)),
    )
    return pl.pallas_call(
        _ffn_body,
        grid_spec=grid_spec,
        out_shape=jax.ShapeDtypeStruct((p_rows, d), jnp.float32),
    )(te, valid, xg, W1.astype(jnp.bfloat16), b1.reshape(e, 1, h),
      W2.astype(jnp.bfloat16), b2.reshape(e, 1, d))


def _sc_combine(y, pos0, pos1, w0):
    p_rows, d = y.shape
    n = pos0.shape[0]
    per_w = n // _NW
    mesh = plsc.VectorSubcoreMesh(core_axis_name="c", subcore_axis_name="s")

    @functools.partial(
        pl.kernel,
        mesh=mesh,
        out_type=jax.ShapeDtypeStruct((n, d), jnp.float32),
        scratch_types=[
            pltpu.VMEM((per_w,), jnp.int32),
            pltpu.VMEM((per_w,), jnp.int32),
            pltpu.VMEM((per_w, 16), jnp.float32),
            pltpu.VMEM((per_w, d), jnp.float32),
            pltpu.VMEM((per_w, d), jnp.float32),
            pltpu.SemaphoreType.DMA,
            pltpu.SemaphoreType.DMA,
        ],
    )
    def k(y_hbm, p0_hbm, p1_hbm, w0_hbm, out_hbm,
          i0_v, i1_v, w0_v, buf0, buf1, sem0, sem1):
        wid = lax.axis_index("s") * _NC + lax.axis_index("c")
        base = wid * per_w
        pltpu.sync_copy(p0_hbm.at[pl.ds(base, per_w)], i0_v)
        pltpu.sync_copy(p1_hbm.at[pl.ds(base, per_w)], i1_v)
        pltpu.sync_copy(w0_hbm.at[pl.ds(base, per_w)], w0_v)
        g0 = pltpu.async_copy(y_hbm.at[i0_v], buf0, sem0)
        g1 = pltpu.async_copy(y_hbm.at[i1_v], buf1, sem1)
        g0.wait()
        g1.wait()
        cols = d // 16

        def body(r, carry):
            wv = w0_v[r, :]
            for c in range(cols):
                y1 = buf1[r, pl.ds(c * 16, 16)]
                y0 = buf0[r, pl.ds(c * 16, 16)]
                buf0[r, pl.ds(c * 16, 16)] = y1 + wv * (y0 - y1)
            return carry

        lax.fori_loop(0, per_w, body, 0)
        pltpu.sync_copy(buf0, out_hbm.at[pl.ds(base, per_w)])

    return k(y, pos0, pos1, w0)


def kernel(x, Wg, bg, W1, b1, W2, b2):
    n, d = x.shape
    e = Wg.shape[1]
    tile = 256
    f = n * _TOPK
    nt = (f - e) // tile + e
    while (nt * tile // _NW) % 8 != 0:
        nt += 1
    p_rows = nt * tile

    weights, indices = _route(x, Wg, bg)

    flat_e = indices.reshape(-1)
    ohi = (flat_e[:, None] == jnp.arange(e, dtype=jnp.int32)[None, :]).astype(jnp.int32)
    ranks_pe = jnp.cumsum(ohi, axis=0) - ohi
    rank = jnp.sum(ranks_pe * ohi, axis=1)
    counts = jnp.sum(ohi, axis=0)
    tiles_pe = (counts + tile - 1) // tile
    tile_start = jnp.concatenate(
        [jnp.zeros((1,), jnp.int32), jnp.cumsum(tiles_pe)[:-1].astype(jnp.int32)])
    group_start = tile_start * tile
    pos_flat = group_start[flat_e] + rank
    token_of_entry = (jnp.arange(f, dtype=jnp.int32) // _TOPK)
    gidx = (jnp.arange(p_rows, dtype=jnp.int32) % n).at[pos_flat].set(token_of_entry)
    tt = jnp.arange(nt, dtype=jnp.int32)
    used = jnp.sum(tiles_pe).astype(jnp.int32)
    valid = (tt < used).astype(jnp.int32)
    tcl = jnp.minimum(tt, used - 1)
    te = (jnp.sum(tile_start[None, :] <= tcl[:, None], axis=1) - 1).astype(jnp.int32)
    pos = pos_flat.reshape(n, _TOPK)

    xg = _sc_gather(x, gidx, p_rows)
    y = _ffn(xg, te, valid, W1, b1, W2, b2, tile, nt)
    w0b = jnp.broadcast_to(weights[:, :1], (n, 16)) + jnp.zeros((n, 16), jnp.float32)
    out = _sc_combine(y, pos[:, 0], pos[:, 1], w0b)
    return out


# speedup vs baseline: 1.4283x; 1.4283x over previous
import functools

import jax
import jax.numpy as jnp
from jax import lax
from jax.experimental import pallas as pl
from jax.experimental.pallas import tpu as pltpu
from jax.experimental.pallas import tpu_sc as plsc

_NC = 2
_NS = 16
_NW = _NC * _NS

_TOPK = 2


def _routing_body(x_ref, wg_ref, bg_ref, i_ref, r0_ref, r1_ref, cnt_ref,
                  w0b_ref, run_ref):
    logits = jnp.dot(x_ref[...], wg_ref[...],
                     preferred_element_type=jnp.float32) + bg_ref[0][None, :]
    bn, e = logits.shape
    iota = lax.broadcasted_iota(jnp.int32, (bn, e), 1)
    m1 = jnp.max(logits, axis=1, keepdims=True)
    i1 = jnp.min(jnp.where(logits == m1, iota, e), axis=1, keepdims=True)
    masked = jnp.where(iota == i1, -jnp.inf, logits)
    m2 = jnp.max(masked, axis=1, keepdims=True)
    i2 = jnp.min(jnp.where(masked == m2, iota, e), axis=1, keepdims=True)
    z = jnp.exp(m2 - m1)
    w2 = z / (1.0 + z)
    i_ref[...] = jnp.concatenate([i1, i2], axis=1).astype(jnp.int32)
    w0b_ref[...] = jnp.broadcast_to(1.0 - w2, w0b_ref.shape)

    @pl.when(pl.program_id(0) == 0)
    def _():
        run_ref[...] = jnp.zeros_like(run_ref)

    oh0 = (iota == i1).astype(jnp.float32)
    oh1 = (iota == i2).astype(jnp.float32)
    both = oh0 + oh1
    rr = lax.broadcasted_iota(jnp.int32, (bn, bn), 0)
    cc = lax.broadcasted_iota(jnp.int32, (bn, bn), 1)
    tri = (rr > cc).astype(jnp.float32)
    pre = jnp.dot(tri, both, preferred_element_type=jnp.float32)
    tot = run_ref[0][None, :] + pre
    r0_ref[...] = jnp.sum(tot * oh0, axis=1, keepdims=True).astype(jnp.int32)
    r1_ref[...] = jnp.sum(tot * oh1, axis=1, keepdims=True).astype(jnp.int32)
    newrun = run_ref[0] + jnp.sum(both, axis=0)
    run_ref[0, :] = newrun
    cnt_ref[...] = newrun[None, :].astype(jnp.int32)


def _route(x, Wg, bg):
    n, d = x.shape
    e = Wg.shape[1]
    bn = 256
    return pl.pallas_call(
        _routing_body,
        grid=(n // bn,),
        in_specs=[
            pl.BlockSpec((bn, d), lambda i: (i, 0)),
            pl.BlockSpec((d, e), lambda i: (0, 0)),
            pl.BlockSpec((1, e), lambda i: (0, 0)),
        ],
        out_specs=[
            pl.BlockSpec((bn, _TOPK), lambda i: (i, 0)),
            pl.BlockSpec((bn, 1), lambda i: (i, 0)),
            pl.BlockSpec((bn, 1), lambda i: (i, 0)),
            pl.BlockSpec((1, e), lambda i: (0, 0)),
            pl.BlockSpec((bn, 16), lambda i: (i, 0)),
        ],
        out_shape=[
            jax.ShapeDtypeStruct((n, _TOPK), jnp.int32),
            jax.ShapeDtypeStruct((n, 1), jnp.int32),
            jax.ShapeDtypeStruct((n, 1), jnp.int32),
            jax.ShapeDtypeStruct((1, e), jnp.int32),
            jax.ShapeDtypeStruct((n, 16), jnp.float32),
        ],
        scratch_shapes=[pltpu.VMEM((1, e), jnp.float32)],
    )(x, Wg, bg.reshape(1, e))


def _sc_dispatch(x, pos0, pos1, p_rows):
    n, d = x.shape
    per_w = n // _NW
    mesh = plsc.VectorSubcoreMesh(core_axis_name="c", subcore_axis_name="s")

    @functools.partial(
        pl.kernel,
        mesh=mesh,
        out_type=jax.ShapeDtypeStruct((p_rows, d), jnp.float32),
        scratch_types=[
            pltpu.VMEM((per_w,), jnp.int32),
            pltpu.VMEM((per_w,), jnp.int32),
            pltpu.VMEM((per_w, d), jnp.float32),
            pltpu.SemaphoreType.DMA,
            pltpu.SemaphoreType.DMA,
        ],
    )
    def k(x_hbm, p0_hbm, p1_hbm, out_hbm, i0_v, i1_v, xbuf, sem0, sem1):
        wid = lax.axis_index("s") * _NC + lax.axis_index("c")
        base = wid * per_w
        pltpu.sync_copy(p0_hbm.at[pl.ds(base, per_w)], i0_v)
        pltpu.sync_copy(p1_hbm.at[pl.ds(base, per_w)], i1_v)
        pltpu.sync_copy(x_hbm.at[pl.ds(base, per_w)], xbuf)
        s0 = pltpu.async_copy(xbuf, out_hbm.at[i0_v], sem0)
        s1 = pltpu.async_copy(xbuf, out_hbm.at[i1_v], sem1)
        s0.wait()
        s1.wait()

    return k(x, pos0, pos1)


def _ffn_body(te_ref, valid_ref, xg_ref, w1_ref, b1_ref, w2_ref, b2_ref,
              y_ref):
    i = pl.program_id(0)

    @pl.when(valid_ref[i] != 0)
    def _():
        h = jnp.dot(xg_ref[...], w1_ref[0],
                    preferred_element_type=jnp.float32) + b1_ref[0]
        h = jnp.maximum(h, 0.0)
        y_ref[...] = jnp.dot(h, w2_ref[0],
                             preferred_element_type=jnp.float32) + b2_ref[0]


def _ffn(xg, te, valid, W1, b1, W2, b2, tile, nt):
    p_rows, d = xg.shape
    e, _, h = W1.shape
    grid_spec = pltpu.PrefetchScalarGridSpec(
        num_scalar_prefetch=2,
        grid=(nt,),
        in_specs=[
            pl.BlockSpec((tile, d), lambda i, te, v: (i, 0)),
            pl.BlockSpec((1, d, h), lambda i, te, v: (te[i], 0, 0)),
            pl.BlockSpec((1, 1, h), lambda i, te, v: (te[i], 0, 0)),
            pl.BlockSpec((1, h, d), lambda i, te, v: (te[i], 0, 0)),
            pl.BlockSpec((1, 1, d), lambda i, te, v: (te[i], 0, 0)),
        ],
        out_specs=pl.BlockSpec((tile, d), lambda i, te, v: (i, 0)),
    )
    return pl.pallas_call(
        _ffn_body,
        grid_spec=grid_spec,
        out_shape=jax.ShapeDtypeStruct((p_rows, d), jnp.float32),
    )(te, valid, xg, W1, b1.reshape(e, 1, h), W2, b2.reshape(e, 1, d))


def _sc_combine(y, pos0, pos1, w0):
    p_rows, d = y.shape
    n = pos0.shape[0]
    per_w = n // _NW
    mesh = plsc.VectorSubcoreMesh(core_axis_name="c", subcore_axis_name="s")

    @functools.partial(
        pl.kernel,
        mesh=mesh,
        out_type=jax.ShapeDtypeStruct((n, d), jnp.float32),
        scratch_types=[
            pltpu.VMEM((per_w,), jnp.int32),
            pltpu.VMEM((per_w,), jnp.int32),
            pltpu.VMEM((per_w, 16), jnp.float32),
            pltpu.VMEM((per_w, d), jnp.float32),
            pltpu.VMEM((per_w, d), jnp.float32),
            pltpu.SemaphoreType.DMA,
            pltpu.SemaphoreType.DMA,
        ],
    )
    def k(y_hbm, p0_hbm, p1_hbm, w0_hbm, out_hbm,
          i0_v, i1_v, w0_v, buf0, buf1, sem0, sem1):
        wid = lax.axis_index("s") * _NC + lax.axis_index("c")
        base = wid * per_w
        pltpu.sync_copy(p0_hbm.at[pl.ds(base, per_w)], i0_v)
        pltpu.sync_copy(p1_hbm.at[pl.ds(base, per_w)], i1_v)
        pltpu.sync_copy(w0_hbm.at[pl.ds(base, per_w)], w0_v)
        g0 = pltpu.async_copy(y_hbm.at[i0_v], buf0, sem0)
        g1 = pltpu.async_copy(y_hbm.at[i1_v], buf1, sem1)
        g0.wait()
        g1.wait()
        cols = d // 16

        def body(r, carry):
            wv = w0_v[r, :]
            for c in range(cols):
                y1 = buf1[r, pl.ds(c * 16, 16)]
                y0 = buf0[r, pl.ds(c * 16, 16)]
                buf0[r, pl.ds(c * 16, 16)] = y1 + wv * (y0 - y1)
            return carry

        lax.fori_loop(0, per_w, body, 0)
        pltpu.sync_copy(buf0, out_hbm.at[pl.ds(base, per_w)])

    return k(y, pos0, pos1, w0)


def kernel(x, Wg, bg, W1, b1, W2, b2):
    n, d = x.shape
    e = Wg.shape[1]
    tile = 256
    f = n * _TOPK
    nt = (f - e) // tile + e
    while (nt * tile // _NW) % 8 != 0:
        nt += 1
    p_rows = nt * tile

    indices, rank0, rank1, counts2d, w0b = _route(x, Wg, bg)

    counts = counts2d[0]
    tiles_pe = (counts + tile - 1) // tile
    tile_start = jnp.concatenate(
        [jnp.zeros((1,), jnp.int32), jnp.cumsum(tiles_pe)[:-1].astype(jnp.int32)])
    group_start = tile_start * tile
    pos0 = group_start[indices[:, 0]] + rank0[:, 0]
    pos1 = group_start[indices[:, 1]] + rank1[:, 0]
    tt = jnp.arange(nt, dtype=jnp.int32)
    used = jnp.sum(tiles_pe).astype(jnp.int32)
    valid = (tt < used).astype(jnp.int32)
    tcl = jnp.minimum(tt, used - 1)
    te = (jnp.sum(tile_start[None, :] <= tcl[:, None], axis=1) - 1).astype(jnp.int32)

    xg = _sc_dispatch(x, pos0, pos1, p_rows)
    y = _ffn(xg, te, valid, W1, b1, W2, b2, tile, nt)
    out = _sc_combine(y, pos0, pos1, w0b)
    return out
